# Initial kernel scaffold; baseline (speedup 1.0000x reference)
import functools

import jax
import jax.numpy as jnp
from jax import lax
from jax.experimental import pallas as pl
from jax.experimental.pallas import tpu as pltpu
from jax.experimental.pallas import tpu_sc as plsc

_NUM_TABLES = 26
_VOCAB = 100000
_EMBED_DIM = 32
_BATCH = 1024
_HIST = 20

_NUM_WORKERS = 32
_TOTAL_BAGS = _NUM_TABLES * _BATCH
_BAGS_PER_WORKER = _TOTAL_BAGS // _NUM_WORKERS
_CHUNK_BAGS = 64
_CHUNKS_PER_WORKER = _BAGS_PER_WORKER // _CHUNK_BAGS
_ROWS_PER_CHUNK = _CHUNK_BAGS * _HIST
_SUB = 128
_NSUB = _ROWS_PER_CHUNK // _SUB
_TOTAL_CHUNKS = _NUM_WORKERS * _CHUNKS_PER_WORKER
_INV_HIST = 1.0 / _HIST


def _sc_body(tab_hbm, idx_hbm, out_hbm, idx_v, rows_v, out_v, sem):
    wid = lax.axis_index("s") * 2 + lax.axis_index("c")

    def chunk_body(g, carry):
        cid = wid * _CHUNKS_PER_WORKER + g
        pltpu.sync_copy(idx_hbm.at[cid], idx_v)
        copies = []
        for k in range(_NSUB):
            copies.append(
                pltpu.async_copy(
                    tab_hbm.at[idx_v.at[k]],
                    rows_v.at[pl.ds(k * _SUB, _SUB)],
                    sem,
                )
            )
        for c in copies:
            c.wait()

        def bag_body(j, carry2):
            r0 = j * _HIST
            acc_lo = rows_v[r0, pl.ds(0, 16)]
            acc_hi = rows_v[r0, pl.ds(16, 16)]
            for h in range(1, _HIST):
                acc_lo = acc_lo + rows_v[r0 + h, pl.ds(0, 16)]
                acc_hi = acc_hi + rows_v[r0 + h, pl.ds(16, 16)]
            out_v[j, pl.ds(0, 16)] = acc_lo * _INV_HIST
            out_v[j, pl.ds(16, 16)] = acc_hi * _INV_HIST
            return carry2

        lax.fori_loop(0, _CHUNK_BAGS, bag_body, 0)
        pltpu.sync_copy(out_v, out_hbm.at[pl.ds(cid * _CHUNK_BAGS, _CHUNK_BAGS)])
        return carry

    lax.fori_loop(0, _CHUNKS_PER_WORKER, chunk_body, 0)


_sc_lookup = functools.partial(
    pl.kernel,
    out_type=jax.ShapeDtypeStruct((_TOTAL_BAGS, _EMBED_DIM), jnp.float32),
    mesh=plsc.VectorSubcoreMesh(core_axis_name="c", subcore_axis_name="s"),
    scratch_types=[
        pltpu.VMEM((_NSUB, _SUB), jnp.int32),
        pltpu.VMEM((_ROWS_PER_CHUNK, _EMBED_DIM), jnp.float32),
        pltpu.VMEM((_CHUNK_BAGS, _EMBED_DIM), jnp.float32),
        pltpu.SemaphoreType.DMA,
    ],
)(_sc_body)


@jax.jit
def kernel(indices, offsets, tables):
    del offsets
    flat_tables = tables.reshape(_NUM_TABLES * _VOCAB, _EMBED_DIM)
    idx3 = indices.reshape(_TOTAL_CHUNKS, _NSUB, _SUB)
    out_flat = _sc_lookup(flat_tables, idx3)
    return (
        out_flat.reshape(_NUM_TABLES, _BATCH, _EMBED_DIM)
        .transpose(1, 0, 2)
        .reshape(_BATCH, _NUM_TABLES * _EMBED_DIM)
    )


# speedup vs baseline: 2.4468x; 2.4468x over previous
import functools

import jax
import jax.numpy as jnp
from jax import lax
from jax.experimental import pallas as pl
from jax.experimental.pallas import tpu as pltpu
from jax.experimental.pallas import tpu_sc as plsc

_NUM_TABLES = 26
_VOCAB = 100000
_EMBED_DIM = 32
_BATCH = 1024
_HIST = 20

_NUM_WORKERS = 32
_TOTAL_BAGS = _NUM_TABLES * _BATCH
_BAGS_PER_WORKER = _TOTAL_BAGS // _NUM_WORKERS
_CHUNK_BAGS = 64
_CHUNKS_PER_WORKER = _BAGS_PER_WORKER // _CHUNK_BAGS
_ROWS_PER_CHUNK = _CHUNK_BAGS * _HIST
_SUB = 128
_NSUB = _ROWS_PER_CHUNK // _SUB
_TOTAL_CHUNKS = _NUM_WORKERS * _CHUNKS_PER_WORKER
_INV_HIST = 1.0 / _HIST


def _sc_body(tab_hbm, idx_hbm, out_hbm, idx_v, rows_v, out_v, sem):
    wid = lax.axis_index("s") * 2 + lax.axis_index("c")

    def chunk_body(g, carry):
        cid = wid * _CHUNKS_PER_WORKER + g
        pltpu.sync_copy(idx_hbm.at[cid], idx_v)
        copies = []
        for k in range(_NSUB):
            copies.append(
                pltpu.async_copy(
                    tab_hbm.at[idx_v.at[k]],
                    rows_v.at[pl.ds(k * _SUB, _SUB)],
                    sem,
                )
            )
        for c in copies:
            c.wait()

        def bag_body(j, carry2):
            r0 = j * _HIST
            acc_lo = rows_v[r0, pl.ds(0, 16)]
            acc_hi = rows_v[r0, pl.ds(16, 16)]
            for h in range(1, _HIST):
                acc_lo = acc_lo + rows_v[r0 + h, pl.ds(0, 16)]
                acc_hi = acc_hi + rows_v[r0 + h, pl.ds(16, 16)]
            out_v[j, pl.ds(0, 16)] = acc_lo * _INV_HIST
            out_v[j, pl.ds(16, 16)] = acc_hi * _INV_HIST
            return carry2

        lax.fori_loop(0, _CHUNK_BAGS, bag_body, 0)
        pltpu.sync_copy(out_v, out_hbm.at[pl.ds(cid * _CHUNK_BAGS, _CHUNK_BAGS)])
        return carry

    lax.fori_loop(0, _CHUNKS_PER_WORKER, chunk_body, 0)


_sc_lookup = functools.partial(
    pl.kernel,
    out_type=jax.ShapeDtypeStruct((_TOTAL_BAGS, _EMBED_DIM), jnp.float32),
    mesh=plsc.VectorSubcoreMesh(core_axis_name="c", subcore_axis_name="s"),
    scratch_types=[
        pltpu.VMEM((_NSUB, _SUB), jnp.int32),
        pltpu.VMEM((_ROWS_PER_CHUNK, _EMBED_DIM), jnp.float32),
        pltpu.VMEM((_CHUNK_BAGS, _EMBED_DIM), jnp.float32),
        pltpu.SemaphoreType.DMA,
    ],
    compiler_params=pltpu.CompilerParams(use_tc_tiling_on_sc=False),
)(_sc_body)


@jax.jit
def kernel(indices, offsets, tables):
    del offsets
    flat_tables = tables.reshape(_NUM_TABLES * _VOCAB, _EMBED_DIM)
    idx3 = indices.reshape(_TOTAL_CHUNKS, _NSUB, _SUB)
    out_flat = _sc_lookup(flat_tables, idx3)
    return (
        out_flat.reshape(_NUM_TABLES, _BATCH, _EMBED_DIM)
        .transpose(1, 0, 2)
        .reshape(_BATCH, _NUM_TABLES * _EMBED_DIM)
    )


# speedup vs baseline: 2.4943x; 1.0194x over previous
import functools

import jax
import jax.numpy as jnp
from jax import lax
from jax.experimental import pallas as pl
from jax.experimental.pallas import tpu as pltpu
from jax.experimental.pallas import tpu_sc as plsc

_NUM_TABLES = 26
_VOCAB = 100000
_EMBED_DIM = 32
_BATCH = 1024
_HIST = 20

_NUM_WORKERS = 32
_TOTAL_BAGS = _NUM_TABLES * _BATCH
_BAGS_PER_WORKER = _TOTAL_BAGS // _NUM_WORKERS
_CHUNK_BAGS = 64
_CHUNKS_PER_WORKER = _BAGS_PER_WORKER // _CHUNK_BAGS
_ROWS_PER_CHUNK = _CHUNK_BAGS * _HIST
_SUB = 128
_NSUB = _ROWS_PER_CHUNK // _SUB
_TOTAL_CHUNKS = _NUM_WORKERS * _CHUNKS_PER_WORKER
_INV_HIST = 1.0 / _HIST


def _sc_body(tab_hbm, idx_hbm, out_hbm, idx_v, rows_v, out_v, sem0, sem1, isem):
    wid = lax.axis_index("s") * 2 + lax.axis_index("c")
    gather_sems = (sem0, sem1)

    def idx_load(g):
        cid = wid * _CHUNKS_PER_WORKER + g
        return pltpu.async_copy(idx_hbm.at[cid], idx_v.at[g % 2], isem)

    def fire(g):
        buf = g % 2
        copies = []
        for k in range(_NSUB):
            copies.append(
                pltpu.async_copy(
                    tab_hbm.at[idx_v.at[buf].at[k]],
                    rows_v.at[buf].at[pl.ds(k * _SUB, _SUB)],
                    gather_sems[buf],
                )
            )
        return copies

    def compute_and_store(g):
        buf = g % 2
        cid = wid * _CHUNKS_PER_WORKER + g

        def bag_body(j, carry2):
            r0 = j * _HIST
            acc_lo = rows_v[buf, r0, pl.ds(0, 16)]
            acc_hi = rows_v[buf, r0, pl.ds(16, 16)]
            for h in range(1, _HIST):
                acc_lo = acc_lo + rows_v[buf, r0 + h, pl.ds(0, 16)]
                acc_hi = acc_hi + rows_v[buf, r0 + h, pl.ds(16, 16)]
            out_v[j, pl.ds(0, 16)] = acc_lo * _INV_HIST
            out_v[j, pl.ds(16, 16)] = acc_hi * _INV_HIST
            return carry2

        lax.fori_loop(0, _CHUNK_BAGS, bag_body, 0)
        pltpu.sync_copy(out_v, out_hbm.at[pl.ds(cid * _CHUNK_BAGS, _CHUNK_BAGS)])

    idx_copies = [None, None]
    gather_copies = [None, None]
    idx_copies[0] = idx_load(0)
    for g in range(_CHUNKS_PER_WORKER + 1):
        if g < _CHUNKS_PER_WORKER:
            idx_copies[g % 2].wait()
            gather_copies[g % 2] = fire(g)
        if g >= 1:
            for c in gather_copies[(g - 1) % 2]:
                c.wait()
            compute_and_store(g - 1)
        if g + 1 < _CHUNKS_PER_WORKER:
            idx_copies[(g + 1) % 2] = idx_load(g + 1)


_sc_lookup = functools.partial(
    pl.kernel,
    out_type=jax.ShapeDtypeStruct((_TOTAL_BAGS, _EMBED_DIM), jnp.float32),
    mesh=plsc.VectorSubcoreMesh(core_axis_name="c", subcore_axis_name="s"),
    scratch_types=[
        pltpu.VMEM((2, _NSUB, _SUB), jnp.int32),
        pltpu.VMEM((2, _ROWS_PER_CHUNK, _EMBED_DIM), jnp.float32),
        pltpu.VMEM((_CHUNK_BAGS, _EMBED_DIM), jnp.float32),
        pltpu.SemaphoreType.DMA,
        pltpu.SemaphoreType.DMA,
        pltpu.SemaphoreType.DMA,
    ],
    compiler_params=pltpu.CompilerParams(use_tc_tiling_on_sc=False),
)(_sc_body)


@jax.jit
def kernel(indices, offsets, tables):
    del offsets
    flat_tables = tables.reshape(_NUM_TABLES * _VOCAB, _EMBED_DIM)
    idx3 = indices.reshape(_TOTAL_CHUNKS, _NSUB, _SUB)
    out_flat = _sc_lookup(flat_tables, idx3)
    return (
        out_flat.reshape(_NUM_TABLES, _BATCH, _EMBED_DIM)
        .transpose(1, 0, 2)
        .reshape(_BATCH, _NUM_TABLES * _EMBED_DIM)
    )


# speedup vs baseline: 2.4971x; 1.0011x over previous
import functools

import jax
import jax.numpy as jnp
from jax import lax
from jax.experimental import pallas as pl
from jax.experimental.pallas import tpu as pltpu
from jax.experimental.pallas import tpu_sc as plsc

_NUM_TABLES = 26
_VOCAB = 100000
_EMBED_DIM = 32
_BATCH = 1024
_HIST = 20

_NUM_WORKERS = 32
_TOTAL_BAGS = _NUM_TABLES * _BATCH
_BAGS_PER_WORKER = _TOTAL_BAGS // _NUM_WORKERS
_CHUNK_BAGS = 64
_CHUNKS_PER_WORKER = _BAGS_PER_WORKER // _CHUNK_BAGS
_ROWS_PER_CHUNK = _CHUNK_BAGS * _HIST
_SUB = 128
_NSUB = _ROWS_PER_CHUNK // _SUB
_TOTAL_CHUNKS = _NUM_WORKERS * _CHUNKS_PER_WORKER
_INV_HIST = 1.0 / _HIST


def _sc_body(tab_hbm, idx_hbm, out_hbm, idx_v, rows_v, out_v, sem0, sem1, isem):
    wid = lax.axis_index("s") * 2 + lax.axis_index("c")
    gather_sems = (sem0, sem1)

    def idx_load(g):
        cid = wid * _CHUNKS_PER_WORKER + g
        return pltpu.async_copy(idx_hbm.at[cid], idx_v.at[g % 2], isem)

    def fire(g):
        buf = g % 2
        return [
            pltpu.async_copy(
                tab_hbm.at[idx_v.at[buf]],
                rows_v.at[buf],
                gather_sems[buf],
            )
        ]

    def compute_and_store(g):
        buf = g % 2
        cid = wid * _CHUNKS_PER_WORKER + g

        def bag_body(j, carry2):
            r0 = j * _HIST
            acc_lo = rows_v[buf, r0, pl.ds(0, 16)]
            acc_hi = rows_v[buf, r0, pl.ds(16, 16)]
            for h in range(1, _HIST):
                acc_lo = acc_lo + rows_v[buf, r0 + h, pl.ds(0, 16)]
                acc_hi = acc_hi + rows_v[buf, r0 + h, pl.ds(16, 16)]
            out_v[j, pl.ds(0, 16)] = acc_lo * _INV_HIST
            out_v[j, pl.ds(16, 16)] = acc_hi * _INV_HIST
            return carry2

        lax.fori_loop(0, _CHUNK_BAGS, bag_body, 0)
        pltpu.sync_copy(out_v, out_hbm.at[pl.ds(cid * _CHUNK_BAGS, _CHUNK_BAGS)])

    idx_copies = [None, None]
    gather_copies = [None, None]
    idx_copies[0] = idx_load(0)
    for g in range(_CHUNKS_PER_WORKER + 1):
        if g < _CHUNKS_PER_WORKER:
            idx_copies[g % 2].wait()
            gather_copies[g % 2] = fire(g)
        if g >= 1:
            for c in gather_copies[(g - 1) % 2]:
                c.wait()
            compute_and_store(g - 1)
        if g + 1 < _CHUNKS_PER_WORKER:
            idx_copies[(g + 1) % 2] = idx_load(g + 1)


_sc_lookup = functools.partial(
    pl.kernel,
    out_type=jax.ShapeDtypeStruct((_TOTAL_BAGS, _EMBED_DIM), jnp.float32),
    mesh=plsc.VectorSubcoreMesh(core_axis_name="c", subcore_axis_name="s"),
    scratch_types=[
        pltpu.VMEM((2, _ROWS_PER_CHUNK), jnp.int32),
        pltpu.VMEM((2, _ROWS_PER_CHUNK, _EMBED_DIM), jnp.float32),
        pltpu.VMEM((_CHUNK_BAGS, _EMBED_DIM), jnp.float32),
        pltpu.SemaphoreType.DMA,
        pltpu.SemaphoreType.DMA,
        pltpu.SemaphoreType.DMA,
    ],
    compiler_params=pltpu.CompilerParams(use_tc_tiling_on_sc=False),
)(_sc_body)


@jax.jit
def kernel(indices, offsets, tables):
    del offsets
    flat_tables = tables.reshape(_NUM_TABLES * _VOCAB, _EMBED_DIM)
    idx3 = indices.reshape(_TOTAL_CHUNKS, _ROWS_PER_CHUNK)
    out_flat = _sc_lookup(flat_tables, idx3)
    return (
        out_flat.reshape(_NUM_TABLES, _BATCH, _EMBED_DIM)
        .transpose(1, 0, 2)
        .reshape(_BATCH, _NUM_TABLES * _EMBED_DIM)
    )


# speedup vs baseline: 2.5080x; 1.0044x over previous
import functools

import jax
import jax.numpy as jnp
from jax import lax
from jax.experimental import pallas as pl
from jax.experimental.pallas import tpu as pltpu
from jax.experimental.pallas import tpu_sc as plsc

_NUM_TABLES = 26
_VOCAB = 100000
_EMBED_DIM = 32
_BATCH = 1024
_HIST = 20

_NUM_WORKERS = 32
_BATCH_PER_WORKER = _BATCH // _NUM_WORKERS
_ROWS_PER_CHUNK = _BATCH_PER_WORKER * _HIST
_OUT_COLS = _NUM_TABLES * _EMBED_DIM
_INV_HIST = 1.0 / _HIST


def _sc_body(tab_hbm, idx_hbm, out_hbm, idx_v, rows_v, out_v, sem0, sem1, isem):
    wid = lax.axis_index("s") * 2 + lax.axis_index("c")
    gather_sems = (sem0, sem1)

    def idx_load(t):
        return pltpu.async_copy(
            idx_hbm.at[t * _NUM_WORKERS + wid], idx_v.at[t % 2], isem
        )

    def fire(t):
        buf = t % 2
        return pltpu.async_copy(
            tab_hbm.at[idx_v.at[buf]], rows_v.at[buf], gather_sems[buf]
        )

    def reduce_chunk(t):
        buf = t % 2
        col = t * _EMBED_DIM

        def bag_body(j, carry):
            r0 = j * _HIST
            acc_lo = rows_v[buf, r0, pl.ds(0, 16)]
            acc_hi = rows_v[buf, r0, pl.ds(16, 16)]
            for h in range(1, _HIST):
                acc_lo = acc_lo + rows_v[buf, r0 + h, pl.ds(0, 16)]
                acc_hi = acc_hi + rows_v[buf, r0 + h, pl.ds(16, 16)]
            out_v[j, pl.ds(col, 16)] = acc_lo * _INV_HIST
            out_v[j, pl.ds(col + 16, 16)] = acc_hi * _INV_HIST
            return carry

        lax.fori_loop(0, _BATCH_PER_WORKER, bag_body, 0)

    idx_copies = [None, None]
    gather_copies = [None, None]
    idx_copies[0] = idx_load(0)
    for t in range(_NUM_TABLES + 1):
        if t < _NUM_TABLES:
            idx_copies[t % 2].wait()
            gather_copies[t % 2] = fire(t)
        if t >= 1:
            gather_copies[(t - 1) % 2].wait()
            reduce_chunk(t - 1)
        if t + 1 < _NUM_TABLES:
            idx_copies[(t + 1) % 2] = idx_load(t + 1)

    pltpu.sync_copy(out_v, out_hbm.at[pl.ds(wid * _BATCH_PER_WORKER, _BATCH_PER_WORKER)])


_sc_lookup = functools.partial(
    pl.kernel,
    out_type=jax.ShapeDtypeStruct((_BATCH, _OUT_COLS), jnp.float32),
    mesh=plsc.VectorSubcoreMesh(core_axis_name="c", subcore_axis_name="s"),
    scratch_types=[
        pltpu.VMEM((2, _ROWS_PER_CHUNK), jnp.int32),
        pltpu.VMEM((2, _ROWS_PER_CHUNK, _EMBED_DIM), jnp.float32),
        pltpu.VMEM((_BATCH_PER_WORKER, _OUT_COLS), jnp.float32),
        pltpu.SemaphoreType.DMA,
        pltpu.SemaphoreType.DMA,
        pltpu.SemaphoreType.DMA,
    ],
    compiler_params=pltpu.CompilerParams(use_tc_tiling_on_sc=False),
)(_sc_body)


@jax.jit
def kernel(indices, offsets, tables):
    del offsets
    flat_tables = tables.reshape(_NUM_TABLES * _VOCAB, _EMBED_DIM)
    idx2 = indices.reshape(_NUM_TABLES * _NUM_WORKERS, _ROWS_PER_CHUNK)
    return _sc_lookup(flat_tables, idx2)
